# Initial kernel scaffold; baseline (speedup 1.0000x reference)
import functools

import jax
import jax.numpy as jnp
from jax import lax
from jax.experimental import pallas as pl
from jax.experimental.pallas import tpu as pltpu
from jax.experimental.pallas import tpu_sc as plsc

N = 10000
E = 320000
D = 128
G = 16

NP = 10240
EP = 327680
NTILES = 32
EDGES_PER_TILE = EP // NTILES
CHUNK = 128
CHUNKS_PER_TILE = EDGES_PER_TILE // CHUNK
GRP = 5
NGRP = CHUNKS_PER_TILE // GRP
ROWS_PER_TILE = NP // 16


def _sc_scatter_body(h_hbm, zeros_hbm, src_hbm, dst_hbm, out_hbm,
                     acc, sidx, didx, rows, sem):
    c = lax.axis_index("c")
    s = lax.axis_index("s")
    wid = s * 2 + c
    r0 = s * ROWS_PER_TILE

    @pl.when(c == 0)
    def _():
        pltpu.sync_copy(h_hbm.at[pl.ds(r0, ROWS_PER_TILE)],
                        acc.at[pl.ds(r0, ROWS_PER_TILE)])

    @pl.when(c == 1)
    def _():
        pltpu.sync_copy(zeros_hbm, acc.at[pl.ds(r0, ROWS_PER_TILE)])

    plsc.subcore_barrier()

    def group(g, carry):
        row0 = wid * CHUNKS_PER_TILE + g * GRP
        pltpu.sync_copy(src_hbm.at[pl.ds(row0, GRP)], sidx)
        pltpu.sync_copy(dst_hbm.at[pl.ds(row0, GRP)], didx)
        cps = [pltpu.async_copy(h_hbm.at[sidx.at[j]], rows.at[j], sem)
               for j in range(GRP)]
        for cp in cps:
            cp.wait()
        for j in range(GRP):
            pltpu.sync_copy(rows.at[j], acc.at[didx.at[j]], add=True)
        return carry

    lax.fori_loop(0, NGRP, group, 0)
    plsc.subcore_barrier()

    pltpu.sync_copy(acc.at[pl.ds(r0, ROWS_PER_TILE)],
                    out_hbm.at[c, pl.ds(r0, ROWS_PER_TILE)])


_sc_scatter = pl.kernel(
    _sc_scatter_body,
    out_type=jax.ShapeDtypeStruct((2, NP, D), jnp.float32),
    mesh=plsc.VectorSubcoreMesh(core_axis_name="c", subcore_axis_name="s"),
    scratch_types=[
        pltpu.VMEM_SHARED((NP, D), jnp.float32),
        pltpu.VMEM((GRP, CHUNK), jnp.int32),
        pltpu.VMEM((GRP, CHUNK), jnp.int32),
        pltpu.VMEM((GRP, CHUNK, D), jnp.float32),
        pltpu.SemaphoreType.DMA,
    ],
)

ROWS_BLK = 1280
NBLK = NP // ROWS_BLK


def _tc_mlp_body(p_ref, w1_ref, b1_ref, w2_ref, b2_ref, o_ref, *, final_relu):
    m = p_ref[0] + p_ref[1]
    t = jnp.maximum(jnp.dot(m, w1_ref[...],
                            preferred_element_type=jnp.float32) + b1_ref[...], 0.0)
    o = jnp.dot(t, w2_ref[...], preferred_element_type=jnp.float32) + b2_ref[...]
    if final_relu:
        o = jnp.maximum(o, 0.0)
    o_ref[...] = o


def _tc_mlp(p, w1, b1, w2, b2, final_relu):
    return pl.pallas_call(
        functools.partial(_tc_mlp_body, final_relu=final_relu),
        grid=(NBLK,),
        in_specs=[
            pl.BlockSpec((2, ROWS_BLK, D), lambda i: (0, i, 0)),
            pl.BlockSpec((D, D), lambda i: (0, 0)),
            pl.BlockSpec((1, D), lambda i: (0, 0)),
            pl.BlockSpec((D, D), lambda i: (0, 0)),
            pl.BlockSpec((1, D), lambda i: (0, 0)),
        ],
        out_specs=pl.BlockSpec((ROWS_BLK, D), lambda i: (i, 0)),
        out_shape=jax.ShapeDtypeStruct((NP, D), jnp.float32),
    )(p, w1, b1, w2, b2)


def _tc_mlp_pool_body(p_ref, w1_ref, b1_ref, w2_ref, b2_ref, batch_ref, o_ref):
    m = p_ref[0] + p_ref[1]
    t = jnp.maximum(jnp.dot(m, w1_ref[...],
                            preferred_element_type=jnp.float32) + b1_ref[...], 0.0)
    o = jnp.dot(t, w2_ref[...], preferred_element_type=jnp.float32) + b2_ref[...]

    @pl.when(pl.program_id(0) == 0)
    def _():
        o_ref[...] = jnp.full((G, D), -jnp.inf, jnp.float32)

    b = batch_ref[0, 0, :]
    neg = jnp.full_like(o, -jnp.inf)
    segs = [jnp.max(jnp.where((b == g)[:, None], o, neg), axis=0)
            for g in range(G)]
    o_ref[...] = jnp.maximum(o_ref[...], jnp.stack(segs))


def _tc_mlp_pool(p, w1, b1, w2, b2, batch3d):
    return pl.pallas_call(
        _tc_mlp_pool_body,
        grid=(NBLK,),
        in_specs=[
            pl.BlockSpec((2, ROWS_BLK, D), lambda i: (0, i, 0)),
            pl.BlockSpec((D, D), lambda i: (0, 0)),
            pl.BlockSpec((1, D), lambda i: (0, 0)),
            pl.BlockSpec((D, D), lambda i: (0, 0)),
            pl.BlockSpec((1, D), lambda i: (0, 0)),
            pl.BlockSpec((1, 1, ROWS_BLK), lambda i: (i, 0, 0)),
        ],
        out_specs=pl.BlockSpec((G, D), lambda i: (0, 0)),
        out_shape=jax.ShapeDtypeStruct((G, D), jnp.float32),
    )(p, w1, b1, w2, b2, batch3d)


def kernel(x, edge_index, batch, W1_0, b1_0, W2_0, b2_0, W1_1, b1_1, W2_1,
           b2_1, W1_2, b1_2, W2_2, b2_2):
    src = edge_index[0]
    dst = edge_index[1]
    pad_e = EP - E
    src_p = jnp.concatenate([src, jnp.zeros((pad_e,), jnp.int32)]).reshape(EP // CHUNK, CHUNK)
    dst_pad = N + (jnp.arange(pad_e, dtype=jnp.int32) % (NP - N))
    dst_p = jnp.concatenate([dst, dst_pad]).reshape(EP // CHUNK, CHUNK)
    x_p = jnp.concatenate([x, jnp.zeros((NP - N, D), jnp.float32)])
    zeros_blk = jnp.zeros((ROWS_PER_TILE, D), jnp.float32)
    batch3d = jnp.concatenate(
        [batch, jnp.full((NP - N,), G, jnp.int32)]).reshape(NBLK, 1, ROWS_BLK)
    b1s = [b1_0.reshape(1, D), b1_1.reshape(1, D), b1_2.reshape(1, D)]
    b2s = [b2_0.reshape(1, D), b2_1.reshape(1, D), b2_2.reshape(1, D)]
    w1s = [W1_0, W1_1, W1_2]
    w2s = [W2_0, W2_1, W2_2]

    h = x_p
    for i in range(2):
        p = _sc_scatter(h, zeros_blk, src_p, dst_p)
        h = _tc_mlp(p, w1s[i], b1s[i], w2s[i], b2s[i], final_relu=True)
    p = _sc_scatter(h, zeros_blk, src_p, dst_p)
    return _tc_mlp_pool(p, w1s[2], b1s[2], w2s[2], b2s[2], batch3d)


# speedup vs baseline: 2.7588x; 2.7588x over previous
import functools

import jax
import jax.numpy as jnp
from jax import lax
from jax.experimental import pallas as pl
from jax.experimental.pallas import tpu as pltpu
from jax.experimental.pallas import tpu_sc as plsc

N = 10000
E = 320000
D = 128
G = 16

NP = 10240
EP = 327680
NTILES = 32
EDGES_PER_TILE = EP // NTILES
CHUNK = 128
CHUNKS_PER_TILE = EDGES_PER_TILE // CHUNK
SUPER = 8
HALF = 2
NGRP = CHUNKS_PER_TILE // SUPER
ROWS_PER_TILE = NP // 16


def _sc_scatter_body(h_hbm, zeros_hbm, src_hbm, dst_hbm, out_hbm,
                     acc, sidx, didx, rows, sem):
    c = lax.axis_index("c")
    s = lax.axis_index("s")
    wid = s * 2 + c
    r0 = s * ROWS_PER_TILE

    @pl.when(c == 0)
    def _():
        pltpu.sync_copy(h_hbm.at[pl.ds(r0, ROWS_PER_TILE)],
                        acc.at[pl.ds(r0, ROWS_PER_TILE)])

    @pl.when(c == 1)
    def _():
        pltpu.sync_copy(zeros_hbm, acc.at[pl.ds(r0, ROWS_PER_TILE)])

    plsc.subcore_barrier()

    def group(g, carry):
        row0 = wid * CHUNKS_PER_TILE + g * SUPER
        pltpu.sync_copy(src_hbm.at[pl.ds(row0, SUPER)], sidx)
        pltpu.sync_copy(dst_hbm.at[pl.ds(row0, SUPER)], didx)
        for half in range(SUPER // HALF):
            cps = [pltpu.async_copy(h_hbm.at[sidx.at[half * HALF + j]],
                                    rows.at[j], sem)
                   for j in range(HALF)]
            for cp in cps:
                cp.wait()
            for j in range(HALF):
                pltpu.sync_copy(rows.at[j],
                                acc.at[didx.at[half * HALF + j]], add=True)
        return carry

    lax.fori_loop(0, NGRP, group, 0)
    plsc.subcore_barrier()

    pltpu.sync_copy(acc.at[pl.ds(r0, ROWS_PER_TILE)],
                    out_hbm.at[c, pl.ds(r0, ROWS_PER_TILE)])


@functools.lru_cache(maxsize=None)
def _get_sc_scatter():
    return pl.kernel(
        _sc_scatter_body,
        out_type=jax.ShapeDtypeStruct((2, NP, D), jnp.float32),
        mesh=plsc.VectorSubcoreMesh(core_axis_name="c", subcore_axis_name="s"),
        scratch_types=[
            pltpu.VMEM_SHARED((NP, D), jnp.float32),
            pltpu.VMEM((SUPER, CHUNK), jnp.int32),
            pltpu.VMEM((SUPER, CHUNK), jnp.int32),
            pltpu.VMEM((HALF, CHUNK, D), jnp.float32),
            pltpu.SemaphoreType.DMA,
        ],
    )


def _sc_scatter(h, zeros_blk, src_p, dst_p):
    return _get_sc_scatter()(h, zeros_blk, src_p, dst_p)

ROWS_BLK = 1280
NBLK = NP // ROWS_BLK


def _tc_mlp_body(p_ref, w1_ref, b1_ref, w2_ref, b2_ref, o_ref, *, final_relu):
    m = p_ref[0] + p_ref[1]
    t = jnp.maximum(jnp.dot(m, w1_ref[...],
                            preferred_element_type=jnp.float32) + b1_ref[...], 0.0)
    o = jnp.dot(t, w2_ref[...], preferred_element_type=jnp.float32) + b2_ref[...]
    if final_relu:
        o = jnp.maximum(o, 0.0)
    o_ref[...] = o


def _tc_mlp(p, w1, b1, w2, b2, final_relu):
    return pl.pallas_call(
        functools.partial(_tc_mlp_body, final_relu=final_relu),
        grid=(NBLK,),
        in_specs=[
            pl.BlockSpec((2, ROWS_BLK, D), lambda i: (0, i, 0)),
            pl.BlockSpec((D, D), lambda i: (0, 0)),
            pl.BlockSpec((1, D), lambda i: (0, 0)),
            pl.BlockSpec((D, D), lambda i: (0, 0)),
            pl.BlockSpec((1, D), lambda i: (0, 0)),
        ],
        out_specs=pl.BlockSpec((ROWS_BLK, D), lambda i: (i, 0)),
        out_shape=jax.ShapeDtypeStruct((NP, D), jnp.float32),
    )(p, w1, b1, w2, b2)


def _tc_mlp_pool_body(bounds_ref, p_ref, w1_ref, b1_ref, w2_ref, b2_ref,
                      o_ref):
    m = p_ref[0] + p_ref[1]
    t = jnp.maximum(jnp.dot(m, w1_ref[...],
                            preferred_element_type=jnp.float32) + b1_ref[...], 0.0)
    o = jnp.dot(t, w2_ref[...], preferred_element_type=jnp.float32) + b2_ref[...]

    @pl.when(pl.program_id(0) == 0)
    def _():
        o_ref[...] = jnp.full((G, D), -jnp.inf, jnp.float32)

    row0 = pl.program_id(0) * ROWS_BLK
    r2 = jax.lax.broadcasted_iota(jnp.int32, (ROWS_BLK, D), 0) + row0
    neg = jnp.full_like(o, -jnp.inf)
    segs = [jnp.max(jnp.where((r2 >= bounds_ref[g]) & (r2 < bounds_ref[g + 1]),
                              o, neg), axis=0)
            for g in range(G)]
    o_ref[...] = jnp.maximum(o_ref[...], jnp.stack(segs))


def _tc_mlp_pool(p, w1, b1, w2, b2, bounds):
    return pl.pallas_call(
        _tc_mlp_pool_body,
        grid=(NBLK,),
        in_specs=[
            pl.BlockSpec(memory_space=pltpu.MemorySpace.SMEM),
            pl.BlockSpec((2, ROWS_BLK, D), lambda i: (0, i, 0)),
            pl.BlockSpec((D, D), lambda i: (0, 0)),
            pl.BlockSpec((1, D), lambda i: (0, 0)),
            pl.BlockSpec((D, D), lambda i: (0, 0)),
            pl.BlockSpec((1, D), lambda i: (0, 0)),
        ],
        out_specs=pl.BlockSpec((G, D), lambda i: (0, 0)),
        out_shape=jax.ShapeDtypeStruct((G, D), jnp.float32),
    )(bounds, p, w1, b1, w2, b2)


def kernel(x, edge_index, batch, W1_0, b1_0, W2_0, b2_0, W1_1, b1_1, W2_1,
           b2_1, W1_2, b1_2, W2_2, b2_2):
    src = edge_index[0]
    dst = edge_index[1]
    pad_e = EP - E
    src_p = jnp.concatenate([src, jnp.zeros((pad_e,), jnp.int32)]).reshape(EP // CHUNK, CHUNK)
    dst_pad = N + (jnp.arange(pad_e, dtype=jnp.int32) % (NP - N))
    dst_p = jnp.concatenate([dst, dst_pad]).reshape(EP // CHUNK, CHUNK)
    x_p = jnp.concatenate([x, jnp.zeros((NP - N, D), jnp.float32)])
    zeros_blk = jnp.zeros((ROWS_PER_TILE, D), jnp.float32)
    bounds = jnp.searchsorted(batch, jnp.arange(G + 1, dtype=jnp.int32)
                              ).astype(jnp.int32)
    b1s = [b1_0.reshape(1, D), b1_1.reshape(1, D), b1_2.reshape(1, D)]
    b2s = [b2_0.reshape(1, D), b2_1.reshape(1, D), b2_2.reshape(1, D)]
    w1s = [W1_0, W1_1, W1_2]
    w2s = [W2_0, W2_1, W2_2]

    h = x_p
    for i in range(2):
        p = _sc_scatter(h, zeros_blk, src_p, dst_p)
        h = _tc_mlp(p, w1s[i], b1s[i], w2s[i], b2s[i], final_relu=True)
    p = _sc_scatter(h, zeros_blk, src_p, dst_p)
    return _tc_mlp_pool(p, w1s[2], b1s[2], w2s[2], b2s[2], bounds)


# speedup vs baseline: 3.2560x; 1.1802x over previous
import functools

import jax
import jax.numpy as jnp
from jax import lax
from jax.experimental import pallas as pl
from jax.experimental.pallas import tpu as pltpu
from jax.experimental.pallas import tpu_sc as plsc

N = 10000
E = 320000
D = 128
G = 16

NP = 10240
EP = 327680
NTILES = 32
EDGES_PER_TILE = EP // NTILES
CHUNK = 64
CHUNKS_PER_TILE = EDGES_PER_TILE // CHUNK
SUPER = 8
SLOTS = 4
NGRP = CHUNKS_PER_TILE // SUPER
ROWS_PER_TILE = NP // 16


def _sc_scatter_body(h_hbm, zeros_hbm, src_hbm, dst_hbm, out_hbm,
                     acc, sidx, didx, rows, gsem, ssem):
    c = lax.axis_index("c")
    s = lax.axis_index("s")
    wid = s * 2 + c
    r0 = s * ROWS_PER_TILE

    @pl.when(c == 0)
    def _():
        pltpu.sync_copy(h_hbm.at[pl.ds(r0, ROWS_PER_TILE)],
                        acc.at[pl.ds(r0, ROWS_PER_TILE)])

    @pl.when(c == 1)
    def _():
        pltpu.sync_copy(zeros_hbm, acc.at[pl.ds(r0, ROWS_PER_TILE)])

    plsc.subcore_barrier()

    def gather(k):
        return pltpu.async_copy(h_hbm.at[sidx.at[k]], rows.at[k % SLOTS],
                                gsem.at[k % SLOTS])

    def scatter(k):
        return pltpu.async_copy(rows.at[k % SLOTS], acc.at[didx.at[k]],
                                ssem.at[k % SLOTS], add=True)

    def group(g, carry):
        row0 = wid * CHUNKS_PER_TILE + g * SUPER
        pltpu.sync_copy(src_hbm.at[pl.ds(row0, SUPER)], sidx)
        pltpu.sync_copy(dst_hbm.at[pl.ds(row0, SUPER)], didx)
        gathers = [gather(k) for k in range(SLOTS - 1)]
        scatters = [None] * SUPER
        for k in range(SUPER):
            gathers[k].wait()
            scatters[k] = scatter(k)
            nxt = k + SLOTS - 1
            if nxt < SUPER:
                if k >= 1:
                    scatters[k - 1].wait()
                gathers.append(gather(nxt))
        for k in range(SUPER - SLOTS, SUPER):
            scatters[k].wait()
        return carry

    lax.fori_loop(0, NGRP, group, 0)
    plsc.subcore_barrier()

    pltpu.sync_copy(acc.at[pl.ds(r0, ROWS_PER_TILE)],
                    out_hbm.at[c, pl.ds(r0, ROWS_PER_TILE)])


@functools.lru_cache(maxsize=None)
def _get_sc_scatter():
    return pl.kernel(
        _sc_scatter_body,
        out_type=jax.ShapeDtypeStruct((2, NP, D), jnp.float32),
        mesh=plsc.VectorSubcoreMesh(core_axis_name="c", subcore_axis_name="s"),
        scratch_types=[
            pltpu.VMEM_SHARED((NP, D), jnp.float32),
            pltpu.VMEM((SUPER, CHUNK), jnp.int32),
            pltpu.VMEM((SUPER, CHUNK), jnp.int32),
            pltpu.VMEM((SLOTS, CHUNK, D), jnp.float32),
            pltpu.SemaphoreType.DMA((SLOTS,)),
            pltpu.SemaphoreType.DMA((SLOTS,)),
        ],
    )


def _sc_scatter(h, zeros_blk, src_p, dst_p):
    return _get_sc_scatter()(h, zeros_blk, src_p, dst_p)

ROWS_BLK = 1280
NBLK = NP // ROWS_BLK


def _tc_mlp_body(p_ref, w1_ref, b1_ref, w2_ref, b2_ref, o_ref, *, final_relu):
    m = p_ref[0] + p_ref[1]
    t = jnp.maximum(jnp.dot(m, w1_ref[...],
                            preferred_element_type=jnp.float32) + b1_ref[...], 0.0)
    o = jnp.dot(t, w2_ref[...], preferred_element_type=jnp.float32) + b2_ref[...]
    if final_relu:
        o = jnp.maximum(o, 0.0)
    o_ref[...] = o


def _tc_mlp(p, w1, b1, w2, b2, final_relu):
    return pl.pallas_call(
        functools.partial(_tc_mlp_body, final_relu=final_relu),
        grid=(NBLK,),
        in_specs=[
            pl.BlockSpec((2, ROWS_BLK, D), lambda i: (0, i, 0)),
            pl.BlockSpec((D, D), lambda i: (0, 0)),
            pl.BlockSpec((1, D), lambda i: (0, 0)),
            pl.BlockSpec((D, D), lambda i: (0, 0)),
            pl.BlockSpec((1, D), lambda i: (0, 0)),
        ],
        out_specs=pl.BlockSpec((ROWS_BLK, D), lambda i: (i, 0)),
        out_shape=jax.ShapeDtypeStruct((NP, D), jnp.float32),
    )(p, w1, b1, w2, b2)


def _tc_mlp_pool_body(bounds_ref, p_ref, w1_ref, b1_ref, w2_ref, b2_ref,
                      o_ref):
    m = p_ref[0] + p_ref[1]
    t = jnp.maximum(jnp.dot(m, w1_ref[...],
                            preferred_element_type=jnp.float32) + b1_ref[...], 0.0)
    o = jnp.dot(t, w2_ref[...], preferred_element_type=jnp.float32) + b2_ref[...]

    @pl.when(pl.program_id(0) == 0)
    def _():
        o_ref[...] = jnp.full((G, D), -jnp.inf, jnp.float32)

    row0 = pl.program_id(0) * ROWS_BLK
    r2 = jax.lax.broadcasted_iota(jnp.int32, (ROWS_BLK, D), 0) + row0
    neg = jnp.full_like(o, -jnp.inf)
    segs = [jnp.max(jnp.where((r2 >= bounds_ref[g]) & (r2 < bounds_ref[g + 1]),
                              o, neg), axis=0)
            for g in range(G)]
    o_ref[...] = jnp.maximum(o_ref[...], jnp.stack(segs))


def _tc_mlp_pool(p, w1, b1, w2, b2, bounds):
    return pl.pallas_call(
        _tc_mlp_pool_body,
        grid=(NBLK,),
        in_specs=[
            pl.BlockSpec(memory_space=pltpu.MemorySpace.SMEM),
            pl.BlockSpec((2, ROWS_BLK, D), lambda i: (0, i, 0)),
            pl.BlockSpec((D, D), lambda i: (0, 0)),
            pl.BlockSpec((1, D), lambda i: (0, 0)),
            pl.BlockSpec((D, D), lambda i: (0, 0)),
            pl.BlockSpec((1, D), lambda i: (0, 0)),
        ],
        out_specs=pl.BlockSpec((G, D), lambda i: (0, 0)),
        out_shape=jax.ShapeDtypeStruct((G, D), jnp.float32),
    )(bounds, p, w1, b1, w2, b2)


def kernel(x, edge_index, batch, W1_0, b1_0, W2_0, b2_0, W1_1, b1_1, W2_1,
           b2_1, W1_2, b1_2, W2_2, b2_2):
    src = edge_index[0]
    dst = edge_index[1]
    pad_e = EP - E
    src_p = jnp.concatenate([src, jnp.zeros((pad_e,), jnp.int32)]).reshape(EP // CHUNK, CHUNK)
    dst_pad = N + (jnp.arange(pad_e, dtype=jnp.int32) % (NP - N))
    dst_p = jnp.concatenate([dst, dst_pad]).reshape(EP // CHUNK, CHUNK)
    x_p = jnp.concatenate([x, jnp.zeros((NP - N, D), jnp.float32)])
    zeros_blk = jnp.zeros((ROWS_PER_TILE, D), jnp.float32)
    bounds = jnp.searchsorted(batch, jnp.arange(G + 1, dtype=jnp.int32)
                              ).astype(jnp.int32)
    b1s = [b1_0.reshape(1, D), b1_1.reshape(1, D), b1_2.reshape(1, D)]
    b2s = [b2_0.reshape(1, D), b2_1.reshape(1, D), b2_2.reshape(1, D)]
    w1s = [W1_0, W1_1, W1_2]
    w2s = [W2_0, W2_1, W2_2]

    h = x_p
    for i in range(2):
        p = _sc_scatter(h, zeros_blk, src_p, dst_p)
        h = _tc_mlp(p, w1s[i], b1s[i], w2s[i], b2s[i], final_relu=True)
    p = _sc_scatter(h, zeros_blk, src_p, dst_p)
    return _tc_mlp_pool(p, w1s[2], b1s[2], w2s[2], b2s[2], bounds)
